# baseline (speedup 1.0000x reference)
import functools

import jax
import jax.numpy as jnp
from jax import lax
from jax.experimental import pallas as pl
from jax.experimental.pallas import tpu as pltpu
from jax.experimental.pallas import tpu_sc as plsc

N_ROWS = 1000000
BATCH = 16384
DIM = 16
NC = 2
NS = 16
NW = NC * NS
BPW = BATCH // NW
NB = 8
RB = BATCH // NB
RCH = 8192
CCH = -(-N_ROWS // RCH)


def _tc_rowmajor_body(ut_ref, it_ref, ou_ref, oi_ref):
    ou_ref[...] = ut_ref[...].T
    oi_ref[...] = it_ref[...].T


_tc_rowmajor = pl.pallas_call(
    _tc_rowmajor_body,
    grid=(CCH,),
    in_specs=[pl.BlockSpec((DIM, RCH), lambda c: (0, c)) for _ in range(2)],
    out_specs=[pl.BlockSpec((RCH, DIM), lambda c: (c, 0)) for _ in range(2)],
    out_shape=[jax.ShapeDtypeStruct((N_ROWS, DIM), jnp.float32)
               for _ in range(2)],
)

_mesh = plsc.VectorSubcoreMesh(core_axis_name="c", subcore_axis_name="s")


@functools.partial(
    pl.kernel,
    mesh=_mesh,
    out_type=[
        jax.ShapeDtypeStruct((BATCH, DIM), jnp.float32),
        jax.ShapeDtypeStruct((BATCH, DIM), jnp.float32),
        jax.ShapeDtypeStruct((BATCH, DIM), jnp.float32),
    ],
    compiler_params=pltpu.CompilerParams(
        needs_layout_passes=False,
        use_tc_tiling_on_sc=False,
    ),
    scratch_types=[
        pltpu.VMEM((BPW,), jnp.int32),
        pltpu.VMEM((BPW,), jnp.int32),
        pltpu.VMEM((BPW,), jnp.int32),
        pltpu.VMEM((BPW, DIM), jnp.float32),
        pltpu.VMEM((BPW, DIM), jnp.float32),
        pltpu.VMEM((BPW, DIM), jnp.float32),
        pltpu.SemaphoreType.DMA,
    ],
)
def _sc_gather(users_hbm, pos_hbm, neg_hbm, ut_hbm, it_hbm,
               out_u, out_p, out_n, iu, ip, ineg, ru, rp, rn, sem):
    wid = lax.axis_index("s") * NC + lax.axis_index("c")
    base = wid * BPW

    pltpu.sync_copy(users_hbm.at[pl.ds(base, BPW)], iu)
    pltpu.sync_copy(pos_hbm.at[pl.ds(base, BPW)], ip)
    pltpu.sync_copy(neg_hbm.at[pl.ds(base, BPW)], ineg)

    c1 = pltpu.async_copy(ut_hbm.at[iu], ru, sem)
    c2 = pltpu.async_copy(it_hbm.at[ip], rp, sem)
    c3 = pltpu.async_copy(it_hbm.at[ineg], rn, sem)
    c1.wait()
    c2.wait()
    c3.wait()

    pltpu.sync_copy(ru, out_u.at[pl.ds(base, BPW)])
    pltpu.sync_copy(rp, out_p.at[pl.ds(base, BPW)])
    pltpu.sync_copy(rn, out_n.at[pl.ds(base, BPW)])


def _tc_loss_body(u_ref, p_ref, n_ref, o_ref):
    i = pl.program_id(0)
    u = u_ref[...]
    sp = jnp.sum(u * p_ref[...], axis=1)
    sn = jnp.sum(u * n_ref[...], axis=1)
    part = jnp.sum(1.0 / (1.0 + jnp.exp(-sp))) + jnp.sum(1.0 / (1.0 + jnp.exp(sn)))

    @pl.when(i == 0)
    def _init():
        o_ref[...] = jnp.zeros_like(o_ref)

    o_ref[...] += (-part / BATCH).reshape(1, 1)


_tc_loss = pl.pallas_call(
    _tc_loss_body,
    grid=(NB,),
    in_specs=[pl.BlockSpec((RB, DIM), lambda i: (i, 0)) for _ in range(3)],
    out_specs=pl.BlockSpec((1, 1), lambda i: (0, 0)),
    out_shape=jax.ShapeDtypeStruct((1, 1), jnp.float32),
)


def kernel(users, pos_items, neg_items, user_table, item_table):
    u = users.astype(jnp.int32)
    p = pos_items.astype(jnp.int32)
    n = neg_items.reshape(-1).astype(jnp.int32)
    ut_lin, it_lin = _tc_rowmajor(user_table.T, item_table.T)
    gu, gp, gn = _sc_gather(u, p, n, ut_lin, it_lin)
    loss = _tc_loss(gu, gp, gn)[0, 0]
    return (loss, loss, jnp.float32(0.0))
